# Initial kernel scaffold; baseline (speedup 1.0000x reference)
import jax
import jax.numpy as jnp
from jax.experimental import pallas as pl


def kernel(inputs, W):
    raise NotImplementedError("write your pallas kernel here")


# speedup vs baseline: 7.6025x; 7.6025x over previous
import functools

import jax
import jax.numpy as jnp
from jax import lax
from jax.experimental import pallas as pl
from jax.experimental.pallas import tpu as pltpu
from jax.experimental.pallas import tpu_sc as plsc

N = 65536
K = 8192
D = 32
TN = 256
NB = N // TN

_COMMITMENT_COST = 0.25
_DIVERGENCE_COST = 1.0


def _argmin_body(x_ref, x2_ref, w_ref, w2_ref, idx_ref, loss_ref):
    m = lax.dot_general(
        x_ref[...], w_ref[...], (((1,), (1,)), ((), ())),
        preferred_element_type=jnp.float32)
    d2 = (x2_ref[...] - 2.0 * m) + w2_ref[...]
    dist = jnp.sqrt(jnp.maximum(d2, 0.0))
    minval = jnp.min(dist, axis=1, keepdims=True)
    iota = lax.broadcasted_iota(jnp.int32, (TN, K), 1)
    idx = jnp.min(jnp.where(dist == minval, iota, K), axis=1)
    idx_ref[...] = idx.reshape(1, 1, TN)

    @pl.when(pl.program_id(0) == 0)
    def _():
        loss_ref[...] = jnp.zeros_like(loss_ref)

    loss_ref[...] += jnp.sum(minval * minval, keepdims=True)


_argmin_call = pl.pallas_call(
    _argmin_body,
    grid=(NB,),
    in_specs=[
        pl.BlockSpec((TN, D), lambda i: (i, 0)),
        pl.BlockSpec((TN, 1), lambda i: (i, 0)),
        pl.BlockSpec((K, D), lambda i: (0, 0)),
        pl.BlockSpec((1, K), lambda i: (0, 0)),
    ],
    out_specs=[
        pl.BlockSpec((1, 1, TN), lambda i: (i, 0, 0)),
        pl.BlockSpec((1, 1), lambda i: (0, 0)),
    ],
    out_shape=[
        jax.ShapeDtypeStruct((NB, 1, TN), jnp.int32),
        jax.ShapeDtypeStruct((1, 1), jnp.float32),
    ],
)

_NC = 2
_NS = 16
_NW = _NC * _NS
_BPW = N // _NW
_CH = 128
_NCH = _BPW // _CH


@functools.cache
def _sc_gather_call():
    @functools.partial(
        pl.kernel,
        out_type=jax.ShapeDtypeStruct((N, D), jnp.float32),
        mesh=plsc.VectorSubcoreMesh(core_axis_name="c", subcore_axis_name="s"),
        scratch_types=[
            pltpu.VMEM((_NCH, _CH), jnp.int32),
            pltpu.VMEM((_BPW, D), jnp.float32),
            pltpu.SemaphoreType.DMA,
        ],
        compiler_params=pltpu.CompilerParams(use_tc_tiling_on_sc=False),
    )
    def _sc_gather(idx_hbm, w_hbm, out_hbm, idx_v, rows_v, sem):
        wid = lax.axis_index("s") * _NC + lax.axis_index("c")
        base = wid * _BPW
        pltpu.sync_copy(idx_hbm.at[wid], idx_v)
        copies = []
        for j in range(_NCH):
            copies.append(pltpu.async_copy(
                w_hbm.at[idx_v.at[j]], rows_v.at[pl.ds(j * _CH, _CH)], sem))
        for c in copies:
            c.wait()
        pltpu.sync_copy(rows_v, out_hbm.at[pl.ds(base, _BPW)])

    return _sc_gather


def kernel(inputs, W):
    x2 = jnp.sum(inputs ** 2, axis=1, keepdims=True)
    w2 = jnp.sum(W ** 2, axis=1)[None, :]
    idx3, losssum = _argmin_call(inputs, x2, W, w2)
    idx_r = idx3.reshape(_NW, _NCH, _CH)
    quantized = _sc_gather_call()(idx_r, W)
    m = losssum[0, 0] / jnp.float32(N * D)
    loss = m * _DIVERGENCE_COST + _COMMITMENT_COST * m
    return (quantized, loss)


# speedup vs baseline: 12.5197x; 1.6468x over previous
import functools

import jax
import jax.numpy as jnp
from jax import lax
from jax.experimental import pallas as pl
from jax.experimental.pallas import tpu as pltpu
from jax.experimental.pallas import tpu_sc as plsc

N = 65536
K = 8192
D = 32
TN = 256
NB = N // TN

_COMMITMENT_COST = 0.25
_DIVERGENCE_COST = 1.0


def _argmin_body(x_ref, x2_ref, w2x_ref, w2_ref, idx_ref, loss_ref):
    m2 = lax.dot_general(
        x_ref[...], w2x_ref[...], (((1,), (1,)), ((), ())),
        preferred_element_type=jnp.float32)
    d2 = (x2_ref[...] - m2) + w2_ref[...]
    v = jnp.maximum(jnp.min(d2, axis=1, keepdims=True), 0.0)
    s = jnp.sqrt(v)
    T = v
    for _ in range(10):
        T2 = lax.bitcast_convert_type(
            lax.bitcast_convert_type(T, jnp.int32) + 1, jnp.float32)
        T = jnp.where(jnp.sqrt(T2) == s, T2, T)
    fiota = lax.broadcasted_iota(jnp.int32, (TN, K), 1).astype(jnp.float32)
    idx_f = jnp.min(jnp.where(d2 <= T, fiota, jnp.float32(K)), axis=1)
    idx_ref[...] = idx_f.astype(jnp.int32).reshape(1, 1, TN)

    @pl.when(pl.program_id(0) == 0)
    def _():
        loss_ref[...] = jnp.zeros_like(loss_ref)

    loss_ref[...] += jnp.sum(v, keepdims=True)


_argmin_call = pl.pallas_call(
    _argmin_body,
    grid=(NB,),
    in_specs=[
        pl.BlockSpec((TN, D), lambda i: (i, 0)),
        pl.BlockSpec((TN, 1), lambda i: (i, 0)),
        pl.BlockSpec((K, D), lambda i: (0, 0)),
        pl.BlockSpec((1, K), lambda i: (0, 0)),
    ],
    out_specs=[
        pl.BlockSpec((1, 1, TN), lambda i: (i, 0, 0)),
        pl.BlockSpec((1, 1), lambda i: (0, 0)),
    ],
    out_shape=[
        jax.ShapeDtypeStruct((NB, 1, TN), jnp.int32),
        jax.ShapeDtypeStruct((1, 1), jnp.float32),
    ],
)

_NC = 2
_NS = 16
_NW = _NC * _NS
_BPW = N // _NW
_CH = 128
_NCH = _BPW // _CH


@functools.cache
def _sc_gather_call():
    @functools.partial(
        pl.kernel,
        out_type=jax.ShapeDtypeStruct((N, D), jnp.float32),
        mesh=plsc.VectorSubcoreMesh(core_axis_name="c", subcore_axis_name="s"),
        scratch_types=[
            pltpu.VMEM((_NCH, _CH), jnp.int32),
            pltpu.VMEM((_BPW, D), jnp.float32),
            pltpu.SemaphoreType.DMA,
        ],
        compiler_params=pltpu.CompilerParams(use_tc_tiling_on_sc=False),
    )
    def _sc_gather(idx_hbm, w_hbm, out_hbm, idx_v, rows_v, sem):
        wid = lax.axis_index("s") * _NC + lax.axis_index("c")
        base = wid * _BPW
        pltpu.sync_copy(idx_hbm.at[wid], idx_v)
        copies = []
        for j in range(_NCH):
            copies.append(pltpu.async_copy(
                w_hbm.at[idx_v.at[j]], rows_v.at[pl.ds(j * _CH, _CH)], sem))
        for c in copies:
            c.wait()
        pltpu.sync_copy(rows_v, out_hbm.at[pl.ds(base, _BPW)])

    return _sc_gather


def kernel(inputs, W):
    x2 = jnp.sum(inputs ** 2, axis=1, keepdims=True)
    w2 = jnp.sum(W ** 2, axis=1)[None, :]
    idx3, losssum = _argmin_call(inputs, x2, W + W, w2)
    idx_r = idx3.reshape(_NW, _NCH, _CH)
    quantized = _sc_gather_call()(idx_r, W)
    m = losssum[0, 0] / jnp.float32(N * D)
    loss = m * _DIVERGENCE_COST + _COMMITMENT_COST * m
    return (quantized, loss)
